# Initial kernel scaffold; baseline (speedup 1.0000x reference)
import jax
import jax.numpy as jnp
from jax.experimental import pallas as pl


def kernel(bow, table):
    raise NotImplementedError("write your pallas kernel here")


# speedup vs baseline: 3.6265x; 3.6265x over previous
import functools

import jax
import jax.numpy as jnp
from jax import lax
from jax.experimental import pallas as pl
from jax.experimental.pallas import tpu as pltpu
from jax.experimental.pallas import tpu_sc as plsc

D = 300
DM = 256
DT = D - DM
N = 1024 * 200
NC = 2
NS = 16
NW = NC * NS
PER_W = N // NW
CH = 128
NCH = PER_W // CH

_mesh = plsc.VectorSubcoreMesh(core_axis_name="c", subcore_axis_name="s")


@functools.partial(
    pl.kernel,
    mesh=_mesh,
    out_type=jax.ShapeDtypeStruct((N, D), jnp.float32),
    scratch_types=[
        pltpu.VMEM((PER_W,), jnp.int32),
        pltpu.VMEM((CH, DM), jnp.float32),
        pltpu.VMEM((CH, DM), jnp.float32),
        pltpu.VMEM((CH, DT), jnp.float32),
        pltpu.VMEM((CH, DT), jnp.float32),
        pltpu.SemaphoreType.DMA,
        pltpu.SemaphoreType.DMA,
        pltpu.SemaphoreType.DMA,
        pltpu.SemaphoreType.DMA,
    ],
)
def _gather(table_hbm, idx_hbm, out_hbm, idx_v, bufm0, bufm1, buft0, buft1,
            semm0, semm1, semt0, semt1):
    wid = lax.axis_index("s") * NC + lax.axis_index("c")
    base = wid * PER_W
    pltpu.sync_copy(idx_hbm.at[pl.ds(base, PER_W)], idx_v)

    def fire(c, bufm, semm, buft, semt):
        pltpu.async_copy(
            table_hbm.at[idx_v.at[pl.ds(c * CH, CH)], pl.ds(0, DM)], bufm, semm)

        def tail_group(g, carry):
            v = idx_v[pl.ds(c * CH + g * 16, 16)]
            for l in range(16):
                r = v[l]
                pltpu.async_copy(
                    table_hbm.at[pl.ds(r, 1), pl.ds(DM, DT)],
                    buft.at[pl.ds(g * 16 + l, 1)], semt)
            return carry

        lax.fori_loop(0, CH // 16, tail_group, 0)

    def drain_and_store(c, bufm, semm, buft, semt):
        pltpu.make_async_copy(
            table_hbm.at[idx_v.at[pl.ds(0, CH)], pl.ds(0, DM)], bufm, semm).wait()

        def tail_wait(i, carry):
            pltpu.make_async_copy(
                table_hbm.at[pl.ds(0, 1), pl.ds(DM, DT)],
                buft.at[pl.ds(i, 1)], semt).wait()
            return carry

        lax.fori_loop(0, CH, tail_wait, 0)
        rows = pl.ds(base + c * CH, CH)
        pltpu.sync_copy(bufm, out_hbm.at[rows, pl.ds(0, DM)])
        pltpu.sync_copy(buft, out_hbm.at[rows, pl.ds(DM, DT)])

    fire(0, bufm0, semm0, buft0, semt0)

    def pair(p, carry):
        c0 = p * 2
        fire(c0 + 1, bufm1, semm1, buft1, semt1)
        drain_and_store(c0, bufm0, semm0, buft0, semt0)

        @pl.when(c0 + 2 < NCH)
        def _():
            fire(c0 + 2, bufm0, semm0, buft0, semt0)

        drain_and_store(c0 + 1, bufm1, semm1, buft1, semt1)
        return carry

    lax.fori_loop(0, NCH // 2, pair, 0)


def kernel(bow, table):
    idx = bow.reshape(N)
    out = _gather(table, idx)
    return out.reshape(bow.shape[0], bow.shape[1], D)


# speedup vs baseline: 4.6113x; 1.2715x over previous
import functools

import jax
import jax.numpy as jnp
from jax import lax
from jax.experimental import pallas as pl
from jax.experimental.pallas import tpu as pltpu
from jax.experimental.pallas import tpu_sc as plsc

V = 3444546
D = 300
DM = 256
DT = D - DM
N = 1024 * 200
NC = 2
NS = 16
NW = NC * NS
PER_W = N // NW
CH = 128
NCH = PER_W // CH

BV = 2048
GT = (V + BV - 1) // BV

_mesh = plsc.VectorSubcoreMesh(core_axis_name="c", subcore_axis_name="s")


def _transpose_body(x_ref, o_ref):
    o_ref[...] = x_ref[...].T


_transpose = pl.pallas_call(
    _transpose_body,
    grid=(GT,),
    in_specs=[pl.BlockSpec((D, BV), lambda i: (0, i))],
    out_specs=pl.BlockSpec((BV, D), lambda i: (i, 0)),
    out_shape=jax.ShapeDtypeStruct((V, D), jnp.float32),
    compiler_params=pltpu.CompilerParams(dimension_semantics=("arbitrary",)),
)


@functools.partial(
    pl.kernel,
    mesh=_mesh,
    out_type=jax.ShapeDtypeStruct((N, D), jnp.float32),
    scratch_types=[
        pltpu.VMEM((PER_W,), jnp.int32),
        pltpu.VMEM((CH, DM), jnp.float32),
        pltpu.VMEM((CH, DM), jnp.float32),
        pltpu.VMEM((CH, DT), jnp.float32),
        pltpu.VMEM((CH, DT), jnp.float32),
        pltpu.SemaphoreType.DMA,
        pltpu.SemaphoreType.DMA,
        pltpu.SemaphoreType.DMA,
        pltpu.SemaphoreType.DMA,
    ],
)
def _gather(table_hbm, idx_hbm, out_hbm, idx_v, bufm0, bufm1, buft0, buft1,
            semm0, semm1, semt0, semt1):
    wid = lax.axis_index("s") * NC + lax.axis_index("c")
    base = wid * PER_W
    pltpu.sync_copy(idx_hbm.at[pl.ds(base, PER_W)], idx_v)

    def fire(c, bufm, semm, buft, semt):
        pltpu.async_copy(
            table_hbm.at[idx_v.at[pl.ds(c * CH, CH)], pl.ds(0, DM)], bufm, semm)

        def tail_group(g, carry):
            v = idx_v[pl.ds(c * CH + g * 16, 16)]
            for l in range(16):
                r = v[l]
                pltpu.async_copy(
                    table_hbm.at[pl.ds(r, 1), pl.ds(DM, DT)],
                    buft.at[pl.ds(g * 16 + l, 1)], semt)
            return carry

        lax.fori_loop(0, CH // 16, tail_group, 0)

    def drain_and_store(c, bufm, semm, buft, semt):
        pltpu.make_async_copy(
            table_hbm.at[idx_v.at[pl.ds(0, CH)], pl.ds(0, DM)], bufm, semm).wait()

        def tail_wait(i, carry):
            pltpu.make_async_copy(
                table_hbm.at[pl.ds(0, 1), pl.ds(DM, DT)],
                buft.at[pl.ds(i, 1)], semt).wait()
            return carry

        lax.fori_loop(0, CH, tail_wait, 0)
        rows = pl.ds(base + c * CH, CH)
        pltpu.sync_copy(bufm, out_hbm.at[rows, pl.ds(0, DM)])
        pltpu.sync_copy(buft, out_hbm.at[rows, pl.ds(DM, DT)])

    fire(0, bufm0, semm0, buft0, semt0)

    def pair(p, carry):
        c0 = p * 2
        fire(c0 + 1, bufm1, semm1, buft1, semt1)
        drain_and_store(c0, bufm0, semm0, buft0, semt0)

        @pl.when(c0 + 2 < NCH)
        def _():
            fire(c0 + 2, bufm0, semm0, buft0, semt0)

        drain_and_store(c0 + 1, bufm1, semm1, buft1, semt1)
        return carry

    lax.fori_loop(0, NCH // 2, pair, 0)


def kernel(bow, table):
    idx = bow.reshape(N)
    table_rm = _transpose(table.T)
    out = _gather(table_rm, idx)
    return out.reshape(bow.shape[0], bow.shape[1], D)


# speedup vs baseline: 4.9718x; 1.0782x over previous
import functools

import jax
import jax.numpy as jnp
from jax import lax
from jax.experimental import pallas as pl
from jax.experimental.pallas import tpu as pltpu
from jax.experimental.pallas import tpu_sc as plsc

V = 3444546
D = 300
DM = 256
DT = D - DM
N = 1024 * 200
NC = 2
NS = 16
NW = NC * NS
PER_W = N // NW
CH = 128
NCH = PER_W // CH

BV = 4096
GT = (V + BV - 1) // BV

_mesh = plsc.VectorSubcoreMesh(core_axis_name="c", subcore_axis_name="s")


def _transpose_body(x_ref, o_ref):
    o_ref[...] = x_ref[...].T


_transpose = pl.pallas_call(
    _transpose_body,
    grid=(GT,),
    in_specs=[pl.BlockSpec((D, BV), lambda i: (0, i))],
    out_specs=pl.BlockSpec((BV, D), lambda i: (i, 0)),
    out_shape=jax.ShapeDtypeStruct((V, D), jnp.float32),
    compiler_params=pltpu.CompilerParams(dimension_semantics=("arbitrary",)),
)


@functools.partial(
    pl.kernel,
    mesh=_mesh,
    out_type=jax.ShapeDtypeStruct((N, D), jnp.float32),
    scratch_types=[
        pltpu.VMEM((PER_W,), jnp.int32),
        pltpu.VMEM((CH, DM), jnp.float32),
        pltpu.VMEM((CH, DM), jnp.float32),
        pltpu.VMEM((CH, DT), jnp.float32),
        pltpu.VMEM((CH, DT), jnp.float32),
        pltpu.SemaphoreType.DMA,
        pltpu.SemaphoreType.DMA,
        pltpu.SemaphoreType.DMA,
        pltpu.SemaphoreType.DMA,
    ],
)
def _gather(table_hbm, idx_hbm, out_hbm, idx_v, bufm0, bufm1, buft0, buft1,
            semm0, semm1, semt0, semt1):
    wid = lax.axis_index("s") * NC + lax.axis_index("c")
    base = wid * PER_W
    pltpu.sync_copy(idx_hbm.at[pl.ds(base, PER_W)], idx_v)

    def fire(c, bufm, semm, buft, semt):
        pltpu.async_copy(
            table_hbm.at[idx_v.at[pl.ds(c * CH, CH)], pl.ds(0, DM)], bufm, semm)

        def tail_group(g, carry):
            v = idx_v[pl.ds(c * CH + g * 16, 16)]
            for l in range(16):
                r = v[l]
                pltpu.async_copy(
                    table_hbm.at[pl.ds(r, 1), pl.ds(DM, DT)],
                    buft.at[pl.ds(g * 16 + l, 1)], semt)
            return carry

        lax.fori_loop(0, CH // 16, tail_group, 0)

    def drain_and_store(c, bufm, semm, buft, semt):
        pltpu.make_async_copy(
            table_hbm.at[idx_v.at[pl.ds(0, CH)], pl.ds(0, DM)], bufm, semm).wait()

        def tail_wait(i, carry):
            pltpu.make_async_copy(
                table_hbm.at[pl.ds(0, 1), pl.ds(DM, DT)],
                buft.at[pl.ds(i, 1)], semt).wait()
            return carry

        lax.fori_loop(0, CH, tail_wait, 0)
        rows = pl.ds(base + c * CH, CH)
        pltpu.sync_copy(bufm, out_hbm.at[rows, pl.ds(0, DM)])
        pltpu.sync_copy(buft, out_hbm.at[rows, pl.ds(DM, DT)])

    fire(0, bufm0, semm0, buft0, semt0)

    def pair(p, carry):
        c0 = p * 2
        fire(c0 + 1, bufm1, semm1, buft1, semt1)
        drain_and_store(c0, bufm0, semm0, buft0, semt0)

        @pl.when(c0 + 2 < NCH)
        def _():
            fire(c0 + 2, bufm0, semm0, buft0, semt0)

        drain_and_store(c0 + 1, bufm1, semm1, buft1, semt1)
        return carry

    lax.fori_loop(0, NCH // 2, pair, 0)


def kernel(bow, table):
    idx = bow.reshape(N)
    table_rm = _transpose(table.T)
    out = _gather(table_rm, idx)
    return out.reshape(bow.shape[0], bow.shape[1], D)


# speedup vs baseline: 5.0494x; 1.0156x over previous
import functools

import jax
import jax.numpy as jnp
from jax import lax
from jax.experimental import pallas as pl
from jax.experimental.pallas import tpu as pltpu
from jax.experimental.pallas import tpu_sc as plsc

V = 3444546
D = 300
DM = 256
DT = D - DM
N = 1024 * 200
NC = 2
NS = 16
NW = NC * NS
PER_W = N // NW
CH = 128
NCH = PER_W // CH

BV = 8192
GT = (V + BV - 1) // BV

_mesh = plsc.VectorSubcoreMesh(core_axis_name="c", subcore_axis_name="s")


def _transpose_body(x_ref, o_ref):
    o_ref[...] = x_ref[...].T


_transpose = pl.pallas_call(
    _transpose_body,
    grid=(GT,),
    in_specs=[pl.BlockSpec((D, BV), lambda i: (0, i))],
    out_specs=pl.BlockSpec((BV, D), lambda i: (i, 0)),
    out_shape=jax.ShapeDtypeStruct((V, D), jnp.float32),
    compiler_params=pltpu.CompilerParams(dimension_semantics=("arbitrary",)),
)


@functools.partial(
    pl.kernel,
    mesh=_mesh,
    out_type=jax.ShapeDtypeStruct((N, D), jnp.float32),
    scratch_types=[
        pltpu.VMEM((PER_W,), jnp.int32),
        pltpu.VMEM((CH, DM), jnp.float32),
        pltpu.VMEM((CH, DM), jnp.float32),
        pltpu.VMEM((CH, DT), jnp.float32),
        pltpu.VMEM((CH, DT), jnp.float32),
        pltpu.SemaphoreType.DMA,
        pltpu.SemaphoreType.DMA,
        pltpu.SemaphoreType.DMA,
        pltpu.SemaphoreType.DMA,
    ],
)
def _gather(table_hbm, idx_hbm, out_hbm, idx_v, bufm0, bufm1, buft0, buft1,
            semm0, semm1, semt0, semt1):
    wid = lax.axis_index("s") * NC + lax.axis_index("c")
    base = wid * PER_W
    pltpu.sync_copy(idx_hbm.at[pl.ds(base, PER_W)], idx_v)

    def fire(c, bufm, semm, buft, semt):
        pltpu.async_copy(
            table_hbm.at[idx_v.at[pl.ds(c * CH, CH)], pl.ds(0, DM)], bufm, semm)

        def tail_group(g, carry):
            v = idx_v[pl.ds(c * CH + g * 16, 16)]
            for l in range(16):
                r = v[l]
                pltpu.async_copy(
                    table_hbm.at[pl.ds(r, 1), pl.ds(DM, DT)],
                    buft.at[pl.ds(g * 16 + l, 1)], semt)
            return carry

        lax.fori_loop(0, CH // 16, tail_group, 0)

    def drain_and_store(c, bufm, semm, buft, semt):
        pltpu.make_async_copy(
            table_hbm.at[idx_v.at[pl.ds(0, CH)], pl.ds(0, DM)], bufm, semm).wait()

        def tail_wait(i, carry):
            pltpu.make_async_copy(
                table_hbm.at[pl.ds(0, 1), pl.ds(DM, DT)],
                buft.at[pl.ds(i, 1)], semt).wait()
            return carry

        lax.fori_loop(0, CH, tail_wait, 0)
        rows = pl.ds(base + c * CH, CH)
        pltpu.sync_copy(bufm, out_hbm.at[rows, pl.ds(0, DM)])
        pltpu.sync_copy(buft, out_hbm.at[rows, pl.ds(DM, DT)])

    fire(0, bufm0, semm0, buft0, semt0)

    def pair(p, carry):
        c0 = p * 2
        fire(c0 + 1, bufm1, semm1, buft1, semt1)
        drain_and_store(c0, bufm0, semm0, buft0, semt0)

        @pl.when(c0 + 2 < NCH)
        def _():
            fire(c0 + 2, bufm0, semm0, buft0, semt0)

        drain_and_store(c0 + 1, bufm1, semm1, buft1, semt1)
        return carry

    lax.fori_loop(0, NCH // 2, pair, 0)


def kernel(bow, table):
    idx = bow.reshape(N)
    table_rm = _transpose(table.T)
    out = _gather(table_rm, idx)
    return out.reshape(bow.shape[0], bow.shape[1], D)


# speedup vs baseline: 5.0647x; 1.0030x over previous
import functools

import jax
import jax.numpy as jnp
from jax import lax
from jax.experimental import pallas as pl
from jax.experimental.pallas import tpu as pltpu
from jax.experimental.pallas import tpu_sc as plsc

V = 3444546
D = 300
DM = 256
DT = D - DM
N = 1024 * 200
NC = 2
NS = 16
NW = NC * NS
PER_W = N // NW
CH = 128
NCH = PER_W // CH

BV = 10240
GT = (V + BV - 1) // BV

_mesh = plsc.VectorSubcoreMesh(core_axis_name="c", subcore_axis_name="s")


def _transpose_body(x_ref, o_ref):
    o_ref[...] = x_ref[...].T


_transpose = pl.pallas_call(
    _transpose_body,
    grid=(GT,),
    in_specs=[pl.BlockSpec((D, BV), lambda i: (0, i))],
    out_specs=pl.BlockSpec((BV, D), lambda i: (i, 0)),
    out_shape=jax.ShapeDtypeStruct((V, D), jnp.float32),
    compiler_params=pltpu.CompilerParams(dimension_semantics=("arbitrary",)),
)


@functools.partial(
    pl.kernel,
    mesh=_mesh,
    out_type=jax.ShapeDtypeStruct((N, D), jnp.float32),
    scratch_types=[
        pltpu.VMEM((PER_W,), jnp.int32),
        pltpu.VMEM((CH, DM), jnp.float32),
        pltpu.VMEM((CH, DM), jnp.float32),
        pltpu.VMEM((CH, DT), jnp.float32),
        pltpu.VMEM((CH, DT), jnp.float32),
        pltpu.SemaphoreType.DMA,
        pltpu.SemaphoreType.DMA,
        pltpu.SemaphoreType.DMA,
        pltpu.SemaphoreType.DMA,
    ],
)
def _gather(table_hbm, idx_hbm, out_hbm, idx_v, bufm0, bufm1, buft0, buft1,
            semm0, semm1, semt0, semt1):
    wid = lax.axis_index("s") * NC + lax.axis_index("c")
    base = wid * PER_W
    pltpu.sync_copy(idx_hbm.at[pl.ds(base, PER_W)], idx_v)

    def fire(c, bufm, semm, buft, semt):
        pltpu.async_copy(
            table_hbm.at[idx_v.at[pl.ds(c * CH, CH)], pl.ds(0, DM)], bufm, semm)

        def tail_group(g, carry):
            v = idx_v[pl.ds(c * CH + g * 16, 16)]
            for l in range(16):
                r = v[l]
                pltpu.async_copy(
                    table_hbm.at[pl.ds(r, 1), pl.ds(DM, DT)],
                    buft.at[pl.ds(g * 16 + l, 1)], semt)
            return carry

        lax.fori_loop(0, CH // 16, tail_group, 0)

    def drain_and_store(c, bufm, semm, buft, semt):
        pltpu.make_async_copy(
            table_hbm.at[idx_v.at[pl.ds(0, CH)], pl.ds(0, DM)], bufm, semm).wait()

        def tail_wait(i, carry):
            pltpu.make_async_copy(
                table_hbm.at[pl.ds(0, 1), pl.ds(DM, DT)],
                buft.at[pl.ds(i, 1)], semt).wait()
            return carry

        lax.fori_loop(0, CH, tail_wait, 0)
        rows = pl.ds(base + c * CH, CH)
        pltpu.sync_copy(bufm, out_hbm.at[rows, pl.ds(0, DM)])
        pltpu.sync_copy(buft, out_hbm.at[rows, pl.ds(DM, DT)])

    fire(0, bufm0, semm0, buft0, semt0)

    def pair(p, carry):
        c0 = p * 2
        fire(c0 + 1, bufm1, semm1, buft1, semt1)
        drain_and_store(c0, bufm0, semm0, buft0, semt0)

        @pl.when(c0 + 2 < NCH)
        def _():
            fire(c0 + 2, bufm0, semm0, buft0, semt0)

        drain_and_store(c0 + 1, bufm1, semm1, buft1, semt1)
        return carry

    lax.fori_loop(0, NCH // 2, pair, 0)


def kernel(bow, table):
    idx = bow.reshape(N)
    table_rm = _transpose(table.T)
    out = _gather(table_rm, idx)
    return out.reshape(bow.shape[0], bow.shape[1], D)
